# baseline (speedup 1.0000x reference)
import math

import jax
import jax.numpy as jnp
from jax.experimental import pallas as pl
from jax.experimental.pallas import tpu as pltpu

_BLK = 512
_HALF_LOG_2PI = 0.5 * math.log(2.0 * math.pi)


def _body(betas_ref, gumbel_ref, eps_ref, muss_ref, stdss_ref, acts_ref, lp_ref,
          w_ref):
    blk, ka = muss_ref.shape
    kk = betas_ref.shape[1]
    aa = ka // kk

    @pl.when(pl.program_id(0) == 0)
    def _():
        row_grp = jax.lax.broadcasted_iota(jnp.int32, (ka, 128), 0) // aa
        col = jax.lax.broadcasted_iota(jnp.int32, (ka, 128), 1)
        w_ref[...] = (row_grp == col).astype(jnp.float32)

    betas = betas_ref[...]
    logits = jnp.log(betas / jnp.sum(betas, axis=-1, keepdims=True))
    score = logits + gumbel_ref[...]

    smax = jnp.max(score, axis=-1, keepdims=True)
    kiota = jax.lax.broadcasted_iota(jnp.int32, score.shape, 1)
    comp = jnp.min(jnp.where(score == smax, kiota, kk), axis=-1, keepdims=True)
    comp_b = jnp.broadcast_to(comp, (blk, aa))

    eps = eps_ref[...]
    mus = muss_ref[...]
    stds = stdss_ref[...]

    acc = mus[:, 0:aa] + stds[:, 0:aa] * eps
    for k in range(1, kk):
        sl = slice(k * aa, (k + 1) * aa)
        a_k = mus[:, sl] + stds[:, sl] * eps
        acc = jnp.where(comp_b == k, a_k, acc)
    acts = jnp.clip(acc, -1.0, 1.0)
    acts_ref[...] = acts

    acts_t = jnp.concatenate([acts] * kk, axis=1)
    z = (acts_t - mus) / stds
    x_terms = -0.5 * z * z - jnp.log(stds)
    log_comp = jax.lax.dot_general(
        x_terms, w_ref[...], (((1,), (0,)), ((), ())),
        preferred_element_type=jnp.float32,
        precision=jax.lax.Precision.HIGHEST,
    )
    x = logits + log_comp[:, :kk] - (aa * _HALF_LOG_2PI)
    m = jnp.max(x, axis=-1, keepdims=True)
    lp = jnp.log(jnp.sum(jnp.exp(x - m), axis=-1, keepdims=True)) + m
    lp_ref[...] = lp


def kernel(muss, stdss, betas):
    b, k, a = muss.shape
    kc, kn = jax.random.split(jax.random.key(42))
    gumbel = jax.random.gumbel(kc, (b, k), muss.dtype)
    eps = jax.random.normal(kn, (b, a), muss.dtype)
    mus2 = muss.reshape(b, k * a)
    stds2 = stdss.reshape(b, k * a)

    grid = (b // _BLK,)
    acts, lp = pl.pallas_call(
        _body,
        grid=grid,
        in_specs=[
            pl.BlockSpec((_BLK, k), lambda i: (i, 0)),
            pl.BlockSpec((_BLK, k), lambda i: (i, 0)),
            pl.BlockSpec((_BLK, a), lambda i: (i, 0)),
            pl.BlockSpec((_BLK, k * a), lambda i: (i, 0)),
            pl.BlockSpec((_BLK, k * a), lambda i: (i, 0)),
        ],
        out_specs=[
            pl.BlockSpec((_BLK, a), lambda i: (i, 0)),
            pl.BlockSpec((_BLK, 1), lambda i: (i, 0)),
        ],
        out_shape=[
            jax.ShapeDtypeStruct((b, a), muss.dtype),
            jax.ShapeDtypeStruct((b, 1), muss.dtype),
        ],
        scratch_shapes=[pltpu.VMEM((k * a, 128), jnp.float32)],
    )(betas, gumbel, eps, mus2, stds2)
    return acts, lp.reshape(b)


# speedup vs baseline: 1.5179x; 1.5179x over previous
import math

import jax
import jax.numpy as jnp
from jax.experimental import pallas as pl

_BLK = 512
_HALF_LOG_2PI = 0.5 * math.log(2.0 * math.pi)


def _body(betas_ref, gumbel_ref, eps_ref, muss_ref, stdss_ref, acts_ref, lp_ref):
    blk, kk, aa = muss_ref.shape

    betas = betas_ref[...]
    logits = jnp.log(betas / jnp.sum(betas, axis=-1, keepdims=True))
    score = logits + gumbel_ref[...]

    smax = jnp.max(score, axis=-1, keepdims=True)
    kiota = jax.lax.broadcasted_iota(jnp.int32, score.shape, 1)
    comp = jnp.min(jnp.where(score == smax, kiota, kk), axis=-1, keepdims=True)

    mus = muss_ref[...]
    stds = stdss_ref[...]
    eps = eps_ref[...]

    cand = mus + stds * eps[:, None, :]
    kiota3 = jax.lax.broadcasted_iota(jnp.int32, cand.shape, 1)
    sel = jnp.sum(jnp.where(kiota3 == comp[:, :, None], cand, 0.0), axis=1)
    acts = jnp.clip(sel, -1.0, 1.0)
    acts_ref[...] = acts

    z = (acts[:, None, :] - mus) / stds
    x_terms = -0.5 * z * z - jnp.log(stds)
    log_comp = jnp.sum(x_terms, axis=-1)
    x = logits + log_comp - (aa * _HALF_LOG_2PI)
    m = jnp.max(x, axis=-1, keepdims=True)
    lp = jnp.log(jnp.sum(jnp.exp(x - m), axis=-1, keepdims=True)) + m
    lp_ref[...] = lp


def kernel(muss, stdss, betas):
    b, k, a = muss.shape
    kc, kn = jax.random.split(jax.random.key(42))
    gumbel = jax.random.gumbel(kc, (b, k), muss.dtype)
    eps = jax.random.normal(kn, (b, a), muss.dtype)

    grid = (b // _BLK,)
    acts, lp = pl.pallas_call(
        _body,
        grid=grid,
        in_specs=[
            pl.BlockSpec((_BLK, k), lambda i: (i, 0)),
            pl.BlockSpec((_BLK, k), lambda i: (i, 0)),
            pl.BlockSpec((_BLK, a), lambda i: (i, 0)),
            pl.BlockSpec((_BLK, k, a), lambda i: (i, 0, 0)),
            pl.BlockSpec((_BLK, k, a), lambda i: (i, 0, 0)),
        ],
        out_specs=[
            pl.BlockSpec((_BLK, a), lambda i: (i, 0)),
            pl.BlockSpec((_BLK, 1), lambda i: (i, 0)),
        ],
        out_shape=[
            jax.ShapeDtypeStruct((b, a), muss.dtype),
            jax.ShapeDtypeStruct((b, 1), muss.dtype),
        ],
    )(betas, gumbel, eps, muss, stdss)
    return acts, lp.reshape(b)


# speedup vs baseline: 1.5777x; 1.0394x over previous
import math

import jax
import jax.numpy as jnp
from jax.experimental import pallas as pl

_BLK = 1024
_HALF_LOG_2PI = 0.5 * math.log(2.0 * math.pi)


def _body(betas_ref, gumbel_ref, eps_ref, muss_ref, stdss_ref, acts_ref, lp_ref):
    blk, kk, aa = muss_ref.shape

    betas = betas_ref[...]
    logits = jnp.log(betas / jnp.sum(betas, axis=-1, keepdims=True))
    score = logits + gumbel_ref[...]

    smax = jnp.max(score, axis=-1, keepdims=True)
    kiota = jax.lax.broadcasted_iota(jnp.int32, score.shape, 1)
    comp = jnp.min(jnp.where(score == smax, kiota, kk), axis=-1, keepdims=True)

    mus = muss_ref[...]
    stds = stdss_ref[...]
    eps = eps_ref[...]

    cand = mus + stds * eps[:, None, :]
    kiota3 = jax.lax.broadcasted_iota(jnp.int32, cand.shape, 1)
    sel = jnp.sum(jnp.where(kiota3 == comp[:, :, None], cand, 0.0), axis=1)
    acts = jnp.clip(sel, -1.0, 1.0)
    acts_ref[...] = acts

    z = (acts[:, None, :] - mus) / stds
    x_terms = -0.5 * z * z - jnp.log(stds)
    log_comp = jnp.sum(x_terms, axis=-1)
    x = logits + log_comp - (aa * _HALF_LOG_2PI)
    m = jnp.max(x, axis=-1, keepdims=True)
    lp = jnp.log(jnp.sum(jnp.exp(x - m), axis=-1, keepdims=True)) + m
    lp_ref[...] = lp


def kernel(muss, stdss, betas):
    b, k, a = muss.shape
    kc, kn = jax.random.split(jax.random.key(42))
    gumbel = jax.random.gumbel(kc, (b, k), muss.dtype)
    eps = jax.random.normal(kn, (b, a), muss.dtype)

    grid = (b // _BLK,)
    acts, lp = pl.pallas_call(
        _body,
        grid=grid,
        in_specs=[
            pl.BlockSpec((_BLK, k), lambda i: (i, 0)),
            pl.BlockSpec((_BLK, k), lambda i: (i, 0)),
            pl.BlockSpec((_BLK, a), lambda i: (i, 0)),
            pl.BlockSpec((_BLK, k, a), lambda i: (i, 0, 0)),
            pl.BlockSpec((_BLK, k, a), lambda i: (i, 0, 0)),
        ],
        out_specs=[
            pl.BlockSpec((_BLK, a), lambda i: (i, 0)),
            pl.BlockSpec((_BLK, 1), lambda i: (i, 0)),
        ],
        out_shape=[
            jax.ShapeDtypeStruct((b, a), muss.dtype),
            jax.ShapeDtypeStruct((b, 1), muss.dtype),
        ],
    )(betas, gumbel, eps, muss, stdss)
    return acts, lp.reshape(b)
